# baseline (speedup 1.0000x reference)
import jax
import jax.numpy as jnp
from jax.experimental import pallas as pl
from jax.experimental.pallas import tpu as pltpu

_CAP_ROWS = 262144
_ROW_DIM = 128
_BLOCK = 4096
_NSLOT = 8
_NBLK = _CAP_ROWS // _BLOCK


def _read(buf_ref, slots, rsems, g):
    return pltpu.make_async_copy(
        buf_ref.at[pl.ds(g * _BLOCK, _BLOCK), :],
        slots.at[g % _NSLOT],
        rsems.at[g % _NSLOT],
    )


def _write(out_ref, slots, wsems, g):
    return pltpu.make_async_copy(
        slots.at[g % _NSLOT],
        out_ref.at[pl.ds(g * _BLOCK, _BLOCK), :],
        wsems.at[g % _NSLOT],
    )


def _pushback_body(end_ref, data_ref, buf_ref, out_ref, slots, rsems, wsems):
    end = end_ref[0]
    end_blk = end // _BLOCK
    local = end % _BLOCK

    for g in range(_NSLOT):
        _read(buf_ref, slots, rsems, g).start()

    for g in range(_NBLK):
        nxt = g + 1
        if nxt < _NBLK and nxt >= _NSLOT:
            _write(out_ref, slots, wsems, nxt - _NSLOT).wait()
            _read(buf_ref, slots, rsems, nxt).start()
        _read(buf_ref, slots, rsems, g).wait()

        @pl.when(end_blk == g)
        def _():
            slots[g % _NSLOT, pl.ds(local, 1), :] = data_ref[...]

        _write(out_ref, slots, wsems, g).start()

    for g in range(_NBLK - _NSLOT, _NBLK):
        _write(out_ref, slots, wsems, g).wait()


def kernel(data, buffer, start_included, end_excluded, length):
    end = jnp.asarray(end_excluded, jnp.int32).reshape(1)
    data2 = data.reshape(1, _ROW_DIM)
    return pl.pallas_call(
        _pushback_body,
        in_specs=[
            pl.BlockSpec(memory_space=pltpu.SMEM),
            pl.BlockSpec(memory_space=pltpu.VMEM),
            pl.BlockSpec(memory_space=pl.ANY),
        ],
        out_specs=pl.BlockSpec(memory_space=pl.ANY),
        out_shape=jax.ShapeDtypeStruct((_CAP_ROWS, _ROW_DIM), jnp.float32),
        scratch_shapes=[
            pltpu.VMEM((_NSLOT, _BLOCK, _ROW_DIM), jnp.float32),
            pltpu.SemaphoreType.DMA((_NSLOT,)),
            pltpu.SemaphoreType.DMA((_NSLOT,)),
        ],
    )(end, data2, buffer)


# speedup vs baseline: 1.0585x; 1.0585x over previous
import jax
import jax.numpy as jnp
from jax.experimental import pallas as pl
from jax.experimental.pallas import tpu as pltpu

_CAP_ROWS = 262144
_ROW_DIM = 128
_BLOCK = 8192
_NSLOT = 6
_NBLK = _CAP_ROWS // _BLOCK


def _read(buf_ref, slots, rsems, g):
    return pltpu.make_async_copy(
        buf_ref.at[pl.ds(g * _BLOCK, _BLOCK), :],
        slots.at[g % _NSLOT],
        rsems.at[g % _NSLOT],
    )


def _write(out_ref, slots, wsems, g):
    return pltpu.make_async_copy(
        slots.at[g % _NSLOT],
        out_ref.at[pl.ds(g * _BLOCK, _BLOCK), :],
        wsems.at[g % _NSLOT],
    )


def _pushback_body(end_ref, data_ref, buf_ref, out_ref, slots, rsems, wsems):
    end = end_ref[0]
    end_blk = end // _BLOCK
    local = end % _BLOCK

    for g in range(_NSLOT):
        _read(buf_ref, slots, rsems, g).start()

    for g in range(_NBLK):
        nxt = g + 1
        if nxt < _NBLK and nxt >= _NSLOT:
            _write(out_ref, slots, wsems, nxt - _NSLOT).wait()
            _read(buf_ref, slots, rsems, nxt).start()
        _read(buf_ref, slots, rsems, g).wait()

        @pl.when(end_blk == g)
        def _():
            slots[g % _NSLOT, pl.ds(local, 1), :] = data_ref[...]

        _write(out_ref, slots, wsems, g).start()

    for g in range(_NBLK - _NSLOT, _NBLK):
        _write(out_ref, slots, wsems, g).wait()


def kernel(data, buffer, start_included, end_excluded, length):
    end = jnp.asarray(end_excluded, jnp.int32).reshape(1)
    data2 = data.reshape(1, _ROW_DIM)
    return pl.pallas_call(
        _pushback_body,
        in_specs=[
            pl.BlockSpec(memory_space=pltpu.SMEM),
            pl.BlockSpec(memory_space=pltpu.VMEM),
            pl.BlockSpec(memory_space=pl.ANY),
        ],
        out_specs=pl.BlockSpec(memory_space=pl.ANY),
        out_shape=jax.ShapeDtypeStruct((_CAP_ROWS, _ROW_DIM), jnp.float32),
        scratch_shapes=[
            pltpu.VMEM((_NSLOT, _BLOCK, _ROW_DIM), jnp.float32),
            pltpu.SemaphoreType.DMA((_NSLOT,)),
            pltpu.SemaphoreType.DMA((_NSLOT,)),
        ],
    )(end, data2, buffer)


# speedup vs baseline: 1.0608x; 1.0021x over previous
import jax
import jax.numpy as jnp
from jax.experimental import pallas as pl
from jax.experimental.pallas import tpu as pltpu

_CAP_ROWS = 262144
_ROW_DIM = 128
_BLOCK = 16384
_NSLOT = 3
_NBLK = _CAP_ROWS // _BLOCK


def _read(buf_ref, slots, rsems, g):
    return pltpu.make_async_copy(
        buf_ref.at[pl.ds(g * _BLOCK, _BLOCK), :],
        slots.at[g % _NSLOT],
        rsems.at[g % _NSLOT],
    )


def _write(out_ref, slots, wsems, g):
    return pltpu.make_async_copy(
        slots.at[g % _NSLOT],
        out_ref.at[pl.ds(g * _BLOCK, _BLOCK), :],
        wsems.at[g % _NSLOT],
    )


def _pushback_body(end_ref, data_ref, buf_ref, out_ref, slots, rsems, wsems):
    end = end_ref[0]
    end_blk = end // _BLOCK
    local = end % _BLOCK

    for g in range(_NSLOT):
        _read(buf_ref, slots, rsems, g).start()

    for g in range(_NBLK):
        nxt = g + 1
        if nxt < _NBLK and nxt >= _NSLOT:
            _write(out_ref, slots, wsems, nxt - _NSLOT).wait()
            _read(buf_ref, slots, rsems, nxt).start()
        _read(buf_ref, slots, rsems, g).wait()

        @pl.when(end_blk == g)
        def _():
            slots[g % _NSLOT, pl.ds(local, 1), :] = data_ref[...]

        _write(out_ref, slots, wsems, g).start()

    for g in range(_NBLK - _NSLOT, _NBLK):
        _write(out_ref, slots, wsems, g).wait()


def kernel(data, buffer, start_included, end_excluded, length):
    end = jnp.asarray(end_excluded, jnp.int32).reshape(1)
    data2 = data.reshape(1, _ROW_DIM)
    return pl.pallas_call(
        _pushback_body,
        in_specs=[
            pl.BlockSpec(memory_space=pltpu.SMEM),
            pl.BlockSpec(memory_space=pltpu.VMEM),
            pl.BlockSpec(memory_space=pl.ANY),
        ],
        out_specs=pl.BlockSpec(memory_space=pl.ANY),
        out_shape=jax.ShapeDtypeStruct((_CAP_ROWS, _ROW_DIM), jnp.float32),
        scratch_shapes=[
            pltpu.VMEM((_NSLOT, _BLOCK, _ROW_DIM), jnp.float32),
            pltpu.SemaphoreType.DMA((_NSLOT,)),
            pltpu.SemaphoreType.DMA((_NSLOT,)),
        ],
    )(end, data2, buffer)


# speedup vs baseline: 1.0627x; 1.0018x over previous
import jax
import jax.numpy as jnp
from jax.experimental import pallas as pl
from jax.experimental.pallas import tpu as pltpu

_CAP_ROWS = 262144
_ROW_DIM = 128
_BLOCK = 32768
_NSLOT = 3
_NBLK = _CAP_ROWS // _BLOCK


def _read(buf_ref, slots, rsems, g):
    return pltpu.make_async_copy(
        buf_ref.at[pl.ds(g * _BLOCK, _BLOCK), :],
        slots.at[g % _NSLOT],
        rsems.at[g % _NSLOT],
    )


def _write(out_ref, slots, wsems, g):
    return pltpu.make_async_copy(
        slots.at[g % _NSLOT],
        out_ref.at[pl.ds(g * _BLOCK, _BLOCK), :],
        wsems.at[g % _NSLOT],
    )


def _pushback_body(end_ref, data_ref, buf_ref, out_ref, slots, rsems, wsems):
    end = end_ref[0]
    end_blk = end // _BLOCK
    local = end % _BLOCK

    for g in range(_NSLOT):
        _read(buf_ref, slots, rsems, g).start()

    for g in range(_NBLK):
        nxt = g + 1
        if nxt < _NBLK and nxt >= _NSLOT:
            _write(out_ref, slots, wsems, nxt - _NSLOT).wait()
            _read(buf_ref, slots, rsems, nxt).start()
        _read(buf_ref, slots, rsems, g).wait()

        @pl.when(end_blk == g)
        def _():
            slots[g % _NSLOT, pl.ds(local, 1), :] = data_ref[...]

        _write(out_ref, slots, wsems, g).start()

    for g in range(_NBLK - _NSLOT, _NBLK):
        _write(out_ref, slots, wsems, g).wait()


def kernel(data, buffer, start_included, end_excluded, length):
    end = jnp.asarray(end_excluded, jnp.int32).reshape(1)
    data2 = data.reshape(1, _ROW_DIM)
    return pl.pallas_call(
        _pushback_body,
        in_specs=[
            pl.BlockSpec(memory_space=pltpu.SMEM),
            pl.BlockSpec(memory_space=pltpu.VMEM),
            pl.BlockSpec(memory_space=pl.ANY),
        ],
        out_specs=pl.BlockSpec(memory_space=pl.ANY),
        out_shape=jax.ShapeDtypeStruct((_CAP_ROWS, _ROW_DIM), jnp.float32),
        scratch_shapes=[
            pltpu.VMEM((_NSLOT, _BLOCK, _ROW_DIM), jnp.float32),
            pltpu.SemaphoreType.DMA((_NSLOT,)),
            pltpu.SemaphoreType.DMA((_NSLOT,)),
        ],
    )(end, data2, buffer)
